# baseline (speedup 1.0000x reference)
import functools

import jax
import jax.numpy as jnp
from jax import lax
from jax.experimental import pallas as pl
from jax.experimental.pallas import tpu as pltpu
from jax.experimental.pallas import tpu_sc as plsc

_V, _D = 1_000_000, 32
_B = 1_048_576
_DP = 128

_NC, _NS = 2, 16
_NW = _NC * _NS
_BPW = _B // _NW
_CHUNK = 256
_NCHUNK = _BPW // _CHUNK
_NBUF = 2


def _body(table_hbm, idx_hbm, out_hbm, idx_all, rows0, rows1, gs0, gs1, ws0, ws1):
    wid = lax.axis_index("s") * _NC + lax.axis_index("c")
    base = wid * _BPW

    pltpu.sync_copy(idx_hbm.at[pl.ds(base, _BPW)], idx_all)

    rows = (rows0, rows1)
    gsem = (gs0, gs1)
    wsem = (ws0, ws1)
    gd = [None] * _NCHUNK
    wd = [None] * _NCHUNK
    for c in range(_NCHUNK):
        b = c % _NBUF
        if c >= _NBUF:
            wd[c - _NBUF].wait()
        gd[c] = pltpu.async_copy(
            table_hbm.at[idx_all.at[pl.ds(c * _CHUNK, _CHUNK)]], rows[b], gsem[b])
        if c >= 1:
            bp = (c - 1) % _NBUF
            gd[c - 1].wait()
            wd[c - 1] = pltpu.async_copy(
                rows[bp].at[:, pl.ds(0, _D)],
                out_hbm.at[pl.ds(base + (c - 1) * _CHUNK, _CHUNK), pl.ds(0, _D)],
                wsem[bp])
    last = _NCHUNK - 1
    gd[last].wait()
    wd[last] = pltpu.async_copy(
        rows[last % _NBUF].at[:, pl.ds(0, _D)],
        out_hbm.at[pl.ds(base + last * _CHUNK, _CHUNK), pl.ds(0, _D)],
        wsem[last % _NBUF])
    wd[last - 1].wait()
    wd[last].wait()


@functools.lru_cache(maxsize=1)
def _build():
    mesh = plsc.VectorSubcoreMesh(core_axis_name="c", subcore_axis_name="s")
    return pl.kernel(
        _body,
        mesh=mesh,
        out_type=jax.ShapeDtypeStruct((_B, _DP), jnp.float32),
        scratch_types=[
            pltpu.VMEM((_BPW,), jnp.int32),
            pltpu.VMEM((_CHUNK, _DP), jnp.float32),
            pltpu.VMEM((_CHUNK, _DP), jnp.float32),
            pltpu.SemaphoreType.DMA,
            pltpu.SemaphoreType.DMA,
            pltpu.SemaphoreType.DMA,
            pltpu.SemaphoreType.DMA,
        ],
        compiler_params=pltpu.CompilerParams(
            use_tc_tiling_on_sc=False, needs_layout_passes=False),
    )


def kernel(features, rev):
    table128 = jnp.pad(features, ((0, 0), (0, _DP - _D)))
    out128 = _build()(table128, rev.astype(jnp.int32))
    return out128[:, :_D]
